# baseline (speedup 1.0000x reference)
import functools

import jax
import jax.numpy as jnp
import numpy as np
from jax import lax
from jax.experimental import pallas as pl
from jax.experimental.pallas import tpu as pltpu
from jax.experimental.pallas import tpu_sc as plsc

VOCAB_N = 100000
HIDDEN_N = 768
MAX_POS_N = 4096
BATCH_N = 4
SEQ_N = 4096

_NC = 2
_NS = 16
_NW = _NC * _NS
_L = 16

_B = BATCH_N * SEQ_N
_PPW = SEQ_N // _NW
_CR = 16
_NB = 4
_NJ = _PPW // _CR
_NK = _NJ * BATCH_N
_NV = HIDDEN_N // _L


@functools.cache
def _scaledsin_table():
    pos = np.arange(MAX_POS_N, dtype=np.float32)
    half_d = HIDDEN_N // 2
    freq_seq = -np.arange(half_d, dtype=np.float32) / np.float32(half_d)
    inv_freq = (np.float32(10000.0) ** freq_seq).astype(np.float32)
    sinusoid = pos[:, None] * inv_freq[None, :]
    tab = np.concatenate([np.sin(sinusoid), np.cos(sinusoid)], axis=-1)
    return jax.device_put(tab.astype(np.float32))


_mesh = plsc.VectorSubcoreMesh(core_axis_name="c", subcore_axis_name="s")


@functools.partial(
    pl.kernel,
    out_type=jax.ShapeDtypeStruct((_B, HIDDEN_N), jnp.float32),
    mesh=_mesh,
    scratch_types=(
        [pltpu.VMEM((BATCH_N * _PPW,), jnp.int32)]
        + [pltpu.VMEM((_CR, HIDDEN_N), jnp.float32)] * (_NB + 2)
        + [pltpu.SemaphoreType.DMA] * (2 * _NB + 2)
    ),
)
def _sc_embed(ids_hbm, tab_hbm, pos_hbm, out_hbm, *scr):
    idx_all = scr[0]
    rows = scr[1:1 + _NB]
    posb = scr[1 + _NB:3 + _NB]
    gsems = scr[3 + _NB:3 + 2 * _NB]
    psems = scr[3 + 2 * _NB:5 + 2 * _NB]
    osems = scr[5 + 2 * _NB:5 + 3 * _NB]

    wid = lax.axis_index("s") * _NC + lax.axis_index("c")
    pbase = wid * _PPW

    def start_pos(j):
        return pltpu.async_copy(pos_hbm.at[pl.ds(pbase + j * _CR, _CR)],
                                posb[j % 2], psems[j % 2])

    pos_pending = start_pos(0)

    for b in range(BATCH_N):
        pltpu.sync_copy(ids_hbm.at[pl.ds(b * SEQ_N + pbase, _PPW)],
                        idx_all.at[pl.ds(b * _PPW, _PPW)])

    def start_gather(kk):
        j, b = divmod(kk, BATCH_N)
        ioff = b * _PPW + j * _CR
        return pltpu.async_copy(tab_hbm.at[idx_all.at[pl.ds(ioff, _CR)]],
                                rows[kk % _NB], gsems[kk % _NB])

    gat_pending = [None] * _NB
    for i in range(min(_NB - 1, _NK)):
        gat_pending[i] = start_gather(i)

    out_pending = [None] * _NB

    for kk in range(_NK):
        j, b = divmod(kk, BATCH_N)
        cur = kk % _NB
        buf = rows[cur]
        pv = posb[j % 2]
        if b == 0:
            pos_pending.wait()
        gat_pending[cur].wait()
        kn = kk + _NB - 1
        if kn < _NK:
            nb = kn % _NB
            if out_pending[nb] is not None:
                out_pending[nb].wait()
                out_pending[nb] = None
            gat_pending[nb] = start_gather(kn)
        if b == 0 and j + 1 < _NJ:
            pos_pending = start_pos(j + 1)

        def row_body(r, c2, buf=buf, pv=pv):
            for v in range(_NV):
                sl = pl.ds(v * _L, _L)
                plsc.addupdate(buf.at[r, sl], pv[r, sl])
            return c2

        lax.fori_loop(0, _CR, row_body, 0)
        cbase = b * SEQ_N + pbase + j * _CR
        out_pending[cur] = pltpu.async_copy(
            buf, out_hbm.at[pl.ds(cbase, _CR)], osems[cur])

    for h in out_pending:
        if h is not None:
            h.wait()


def kernel(input_ids, word_embeddings, scale):
    ids_flat = input_ids.reshape(-1).astype(jnp.int32)
    posemb = _scaledsin_table() * scale.astype(jnp.float32)[0]
    out = _sc_embed(ids_flat, word_embeddings, posemb)
    return out.reshape(BATCH_N, SEQ_N, HIDDEN_N)


# speedup vs baseline: 1.1975x; 1.1975x over previous
import functools

import jax
import jax.numpy as jnp
import numpy as np
from jax import lax
from jax.experimental import pallas as pl
from jax.experimental.pallas import tpu as pltpu
from jax.experimental.pallas import tpu_sc as plsc

VOCAB_N = 100000
HIDDEN_N = 768
MAX_POS_N = 4096
BATCH_N = 4
SEQ_N = 4096

_NC = 2
_NS = 16
_NW = _NC * _NS
_L = 16

_B = BATCH_N * SEQ_N
_PPW = SEQ_N // _NW
_CR = 32
_NB = 3
_NJ = _PPW // _CR
_NK = _NJ * BATCH_N
_NV = HIDDEN_N // _L


@functools.cache
def _scaledsin_table():
    pos = np.arange(MAX_POS_N, dtype=np.float32)
    half_d = HIDDEN_N // 2
    freq_seq = -np.arange(half_d, dtype=np.float32) / np.float32(half_d)
    inv_freq = (np.float32(10000.0) ** freq_seq).astype(np.float32)
    sinusoid = pos[:, None] * inv_freq[None, :]
    tab = np.concatenate([np.sin(sinusoid), np.cos(sinusoid)], axis=-1)
    return jax.device_put(tab.astype(np.float32))


_mesh = plsc.VectorSubcoreMesh(core_axis_name="c", subcore_axis_name="s")


@functools.partial(
    pl.kernel,
    out_type=jax.ShapeDtypeStruct((_B, HIDDEN_N), jnp.float32),
    mesh=_mesh,
    scratch_types=(
        [pltpu.VMEM((BATCH_N, _PPW), jnp.int32)]
        + [pltpu.VMEM((_CR, HIDDEN_N), jnp.float32)] * (_NB + 2)
        + [pltpu.SemaphoreType.DMA] * (2 * _NB + 2)
    ),
)
def _sc_embed(ids_hbm, tab_hbm, pos_hbm, out_hbm, *scr):
    idx_all = scr[0]
    rows = scr[1:1 + _NB]
    posb = scr[1 + _NB:3 + _NB]
    gsems = scr[3 + _NB:3 + 2 * _NB]
    psems = scr[3 + 2 * _NB:5 + 2 * _NB]
    osems = scr[5 + 2 * _NB:5 + 3 * _NB]

    wid = lax.axis_index("s") * _NC + lax.axis_index("c")
    pbase = wid * _PPW

    def start_pos(j):
        return pltpu.async_copy(pos_hbm.at[pl.ds(pbase + j * _CR, _CR)],
                                posb[j % 2], psems[j % 2])

    pos_pending = start_pos(0)

    pltpu.sync_copy(ids_hbm.at[:, pl.ds(pbase, _PPW)], idx_all)

    def start_gather(kk):
        j, b = divmod(kk, BATCH_N)
        idx_ref = idx_all.at[b, pl.ds(j * _CR, _CR)]
        return pltpu.async_copy(tab_hbm.at[idx_ref],
                                rows[kk % _NB], gsems[kk % _NB])

    gat_pending = [None] * _NB
    for i in range(min(_NB - 1, _NK)):
        gat_pending[i] = start_gather(i)

    out_pending = [None] * _NB

    for kk in range(_NK):
        j, b = divmod(kk, BATCH_N)
        cur = kk % _NB
        buf = rows[cur]
        pv = posb[j % 2]
        if b == 0:
            pos_pending.wait()
        gat_pending[cur].wait()
        kn = kk + _NB - 1
        if kn < _NK:
            nb = kn % _NB
            if out_pending[nb] is not None:
                out_pending[nb].wait()
                out_pending[nb] = None
            gat_pending[nb] = start_gather(kn)
        if b == 0 and j + 1 < _NJ:
            pos_pending = start_pos(j + 1)

        def row_body(r, c2, buf=buf, pv=pv):
            for v in range(_NV):
                sl = pl.ds(v * _L, _L)
                plsc.addupdate(buf.at[r, sl], pv[r, sl])
            return c2

        lax.fori_loop(0, _CR, row_body, 0)
        cbase = b * SEQ_N + pbase + j * _CR
        out_pending[cur] = pltpu.async_copy(
            buf, out_hbm.at[pl.ds(cbase, _CR)], osems[cur])

    for h in out_pending:
        if h is not None:
            h.wait()


def kernel(input_ids, word_embeddings, scale):
    ids2d = input_ids.astype(jnp.int32)
    posemb = _scaledsin_table() * scale.astype(jnp.float32)[0]
    out = _sc_embed(ids2d, word_embeddings, posemb)
    return out.reshape(BATCH_N, SEQ_N, HIDDEN_N)


# speedup vs baseline: 1.1991x; 1.0013x over previous
import functools

import jax
import jax.numpy as jnp
import numpy as np
from jax import lax
from jax.experimental import pallas as pl
from jax.experimental.pallas import tpu as pltpu
from jax.experimental.pallas import tpu_sc as plsc

VOCAB_N = 100000
HIDDEN_N = 768
MAX_POS_N = 4096
BATCH_N = 4
SEQ_N = 4096

_NC = 2
_NS = 16
_NW = _NC * _NS
_L = 16

_B = BATCH_N * SEQ_N
_PPW = SEQ_N // _NW
_CR = 32
_NB = 3
_NJ = _PPW // _CR
_NK = _NJ * BATCH_N
_NV = HIDDEN_N // _L


@functools.cache
def _scaledsin_table():
    pos = np.arange(MAX_POS_N, dtype=np.float32)
    half_d = HIDDEN_N // 2
    freq_seq = -np.arange(half_d, dtype=np.float32) / np.float32(half_d)
    inv_freq = (np.float32(10000.0) ** freq_seq).astype(np.float32)
    sinusoid = pos[:, None] * inv_freq[None, :]
    tab = np.concatenate([np.sin(sinusoid), np.cos(sinusoid)], axis=-1)
    return jax.device_put(tab.astype(np.float32))


_mesh = plsc.VectorSubcoreMesh(core_axis_name="c", subcore_axis_name="s")


@functools.partial(
    pl.kernel,
    out_type=jax.ShapeDtypeStruct((_B, HIDDEN_N), jnp.float32),
    mesh=_mesh,
    scratch_types=(
        [pltpu.VMEM((BATCH_N, _PPW), jnp.int32)]
        + [pltpu.VMEM((_CR, HIDDEN_N), jnp.float32)] * (_NB + 2)
        + [pltpu.SemaphoreType.DMA] * (3 * _NB + 2)
    ),
)
def _sc_embed(ids_hbm, tab_hbm, pos_hbm, out_hbm, *scr):
    idx_all = scr[0]
    rows = scr[1:1 + _NB]
    posb = scr[1 + _NB:3 + _NB]
    gsems = scr[3 + _NB:3 + 3 * _NB]
    psems = scr[3 + 3 * _NB:5 + 3 * _NB]
    osems = scr[5 + 3 * _NB:5 + 4 * _NB]

    wid = lax.axis_index("s") * _NC + lax.axis_index("c")
    pbase = wid * _PPW

    def start_pos(j):
        return pltpu.async_copy(pos_hbm.at[pl.ds(pbase + j * _CR, _CR)],
                                posb[j % 2], psems[j % 2])

    pos_pending = start_pos(0)

    pltpu.sync_copy(ids_hbm.at[:, pl.ds(pbase, _PPW)], idx_all)

    _H = _CR // 2

    def start_gather(kk):
        j, b = divmod(kk, BATCH_N)
        buf = rows[kk % _NB]
        h0 = pltpu.async_copy(
            tab_hbm.at[idx_all.at[b, pl.ds(j * _CR, _H)]],
            buf.at[pl.ds(0, _H)], gsems[2 * (kk % _NB)])
        h1 = pltpu.async_copy(
            tab_hbm.at[idx_all.at[b, pl.ds(j * _CR + _H, _H)]],
            buf.at[pl.ds(_H, _H)], gsems[2 * (kk % _NB) + 1])
        return (h0, h1)

    gat_pending = [None] * _NB
    for i in range(min(_NB - 1, _NK)):
        gat_pending[i] = start_gather(i)

    out_pending = [None] * _NB

    for kk in range(_NK):
        j, b = divmod(kk, BATCH_N)
        cur = kk % _NB
        buf = rows[cur]
        pv = posb[j % 2]
        if b == 0:
            pos_pending.wait()
        gat_pending[cur][0].wait()
        gat_pending[cur][1].wait()
        kn = kk + _NB - 1
        if kn < _NK:
            nb = kn % _NB
            if out_pending[nb] is not None:
                out_pending[nb].wait()
                out_pending[nb] = None
            gat_pending[nb] = start_gather(kn)
        if b == 0 and j + 1 < _NJ:
            pos_pending = start_pos(j + 1)

        def row_body(r, c2, buf=buf, pv=pv):
            for v in range(_NV):
                sl = pl.ds(v * _L, _L)
                plsc.addupdate(buf.at[r, sl], pv[r, sl])
            return c2

        lax.fori_loop(0, _CR, row_body, 0)
        cbase = b * SEQ_N + pbase + j * _CR
        out_pending[cur] = pltpu.async_copy(
            buf, out_hbm.at[pl.ds(cbase, _CR)], osems[cur])

    for h in out_pending:
        if h is not None:
            h.wait()


def kernel(input_ids, word_embeddings, scale):
    ids2d = input_ids.astype(jnp.int32)
    posemb = _scaledsin_table() * scale.astype(jnp.float32)[0]
    out = _sc_embed(ids2d, word_embeddings, posemb)
    return out.reshape(BATCH_N, SEQ_N, HIDDEN_N)
